# Initial kernel scaffold; baseline (speedup 1.0000x reference)
import jax
import jax.numpy as jnp
from jax.experimental import pallas as pl


def kernel(edge_index, edge_attr, params):
    raise NotImplementedError("write your pallas kernel here")


# speedup vs baseline: 1.3471x; 1.3471x over previous
import functools

import jax
import jax.numpy as jnp
from jax import lax
from jax.experimental import pallas as pl
from jax.experimental.pallas import tpu as pltpu
from jax.experimental.pallas import tpu_sc as plsc

N = 10000
E = 320000
D = 128
ED = 16
NC = 2
NS = 16
NW = NC * NS
CHUNK = 128
CPW = 80
NCH2 = NW * CPW
E2 = NCH2 * CHUNK
NEG = -1e30
RPT = 624
REM = N - NS * RPT


@functools.cache
def _mesh():
    return plsc.VectorSubcoreMesh(
        core_axis_name="c", subcore_axis_name="s", num_cores=NC, num_subcores=NS)


def _zero_spmem(zsrc, sh, s):
    r0 = s * RPT
    for t in range(4):
        pltpu.sync_copy(zsrc, sh.at[pl.ds(r0 + t * CHUNK, CHUNK)])
    pltpu.sync_copy(zsrc.at[pl.ds(0, RPT - 4 * CHUNK)],
                    sh.at[pl.ds(r0 + 4 * CHUNK, RPT - 4 * CHUNK)])

    @pl.when(s == NS - 1)
    def _():
        pltpu.sync_copy(zsrc.at[pl.ds(0, REM)], sh.at[pl.ds(NS * RPT, REM)])


def _writeout(sh, out, c, s):
    r0 = s * RPT
    pltpu.sync_copy(sh.at[pl.ds(r0, RPT)], out.at[c].at[pl.ds(r0, RPT)])

    @pl.when(s == NS - 1)
    def _():
        pltpu.sync_copy(sh.at[pl.ds(NS * RPT, REM)],
                        out.at[c].at[pl.ds(NS * RPT, REM)])


def _fill_const(buf, nrows, val):
    v = jnp.full((16,), val, jnp.float32)

    def body(i, _):
        for j in range(D // 16):
            buf[i, pl.ds(j * 16, 16)] = v
        return 0
    lax.fori_loop(0, nrows, body, 0)


def _sc_counts(dstc1d):
    @functools.partial(
        pl.kernel,
        out_type=pltpu.HBM((NC, N, D), jnp.float32),
        mesh=_mesh(),
        scratch_types=[
            pltpu.VMEM((CHUNK,), jnp.int32),
            pltpu.VMEM((CHUNK, D), jnp.float32),
            pltpu.VMEM((CHUNK, D), jnp.float32),
            pltpu.VMEM_SHARED((N + 16, D), jnp.float32),
        ],
    )
    def k(dst_hbm, c_out, idx_d, ones, zc, c_sh):
        c = lax.axis_index("c")
        s = lax.axis_index("s")
        w = s * NC + c
        lo = w * CPW

        _fill_const(ones, CHUNK, 1.0)
        _fill_const(zc, CHUNK, 0.0)
        _zero_spmem(zc, c_sh, s)
        plsc.subcore_barrier()

        def cb(g, _):
            pltpu.sync_copy(dst_hbm.at[pl.ds((lo + g) * CHUNK, CHUNK)], idx_d)
            pltpu.sync_copy(ones, c_sh.at[idx_d], add=True)
            return 0
        lax.fori_loop(0, CPW, cb, 0)

        plsc.subcore_barrier()
        _writeout(c_sh, c_out, c, s)

    return k(dstc1d)


def _sc_pass0(eb0, dst1d, dep):
    @functools.partial(
        pl.kernel,
        out_type=pltpu.HBM((NC, N, D), jnp.float32),
        mesh=_mesh(),
        scratch_types=[
            pltpu.VMEM((CHUNK,), jnp.int32),
            pltpu.VMEM((2 * CHUNK, D), jnp.float32),
            pltpu.VMEM_SHARED((N, D), jnp.float32),
            pltpu.SemaphoreType.DMA,
        ],
    )
    def k(eb_hbm, dst_hbm, dep_hbm, s_out, idx_d, ebuf, s_sh, esem):
        c = lax.axis_index("c")
        s = lax.axis_index("s")
        w = s * NC + c
        lo = w * CPW

        _fill_const(ebuf, CHUNK, 0.0)
        _zero_spmem(ebuf.at[pl.ds(0, CHUNK)], s_sh, s)
        plsc.subcore_barrier()

        def issue(g, b):
            pltpu.async_copy(eb_hbm.at[pl.ds((lo + g) * CHUNK, CHUNK)],
                             ebuf.at[pl.ds(b * CHUNK, CHUNK)], esem)

        issue(0, 0)

        def body(g, _):
            b = lax.rem(g, 2)
            pltpu.sync_copy(dst_hbm.at[pl.ds((lo + g) * CHUNK, CHUNK)], idx_d)
            pltpu.make_async_copy(eb_hbm.at[pl.ds((lo + g) * CHUNK, CHUNK)],
                                  ebuf.at[pl.ds(b * CHUNK, CHUNK)],
                                  esem).wait()

            @pl.when(g + 1 < CPW)
            def _():
                issue(g + 1, 1 - b)
            pltpu.sync_copy(ebuf.at[pl.ds(b * CHUNK, CHUNK)],
                            s_sh.at[idx_d], add=True)
            return 0
        lax.fori_loop(0, CPW, body, 0)

        plsc.subcore_barrier()
        _writeout(s_sh, s_out, c, s)

    return k(eb0, dst1d, dep)


def _sc_pass(y, eb, src1d, dst1d):
    @functools.partial(
        pl.kernel,
        out_type=pltpu.HBM((NC, N, D), jnp.float32),
        mesh=_mesh(),
        scratch_types=[
            pltpu.VMEM((CHUNK,), jnp.int32),
            pltpu.VMEM((CHUNK,), jnp.int32),
            pltpu.VMEM((CHUNK, D), jnp.float32),
            pltpu.VMEM((2 * CHUNK, D), jnp.float32),
            pltpu.VMEM_SHARED((N, D), jnp.float32),
            pltpu.SemaphoreType.DMA,
            pltpu.SemaphoreType.DMA,
        ],
    )
    def k(y_hbm, eb_hbm, src_hbm, dst_hbm, s_out,
          idx_g, idx_d, rows, ebuf, s_sh, gsem, esem):
        c = lax.axis_index("c")
        s = lax.axis_index("s")
        w = s * NC + c
        lo = w * CPW

        _fill_const(ebuf, CHUNK, 0.0)
        _zero_spmem(ebuf.at[pl.ds(0, CHUNK)], s_sh, s)
        plsc.subcore_barrier()

        pltpu.async_copy(eb_hbm.at[pl.ds(lo * CHUNK, CHUNK)],
                         ebuf.at[pl.ds(0, CHUNK)], esem)

        def body(g, _):
            b = lax.rem(g, 2)
            r0b = b * CHUNK
            pltpu.sync_copy(src_hbm.at[pl.ds((lo + g) * CHUNK, CHUNK)], idx_g)
            pltpu.sync_copy(dst_hbm.at[pl.ds((lo + g) * CHUNK, CHUNK)], idx_d)
            pltpu.async_copy(y_hbm.at[idx_g], rows, gsem)

            @pl.when(g + 1 < CPW)
            def _():
                pltpu.async_copy(eb_hbm.at[pl.ds((lo + g + 1) * CHUNK, CHUNK)],
                                 ebuf.at[pl.ds((1 - b) * CHUNK, CHUNK)], esem)

            pltpu.make_async_copy(y_hbm.at[idx_g], rows, gsem).wait()
            pltpu.make_async_copy(eb_hbm.at[pl.ds((lo + g) * CHUNK, CHUNK)],
                                  ebuf.at[pl.ds(r0b, CHUNK)], esem).wait()

            def rbody(i, _):
                for j in range(D // 16):
                    sl = pl.ds(j * 16, 16)
                    rows[i, sl] = jnp.maximum(
                        rows[i, sl] + ebuf[r0b + i, sl], 0.0)
                return 0
            lax.fori_loop(0, CHUNK, rbody, 0)

            pltpu.sync_copy(rows, s_sh.at[idx_d], add=True)
            return 0
        lax.fori_loop(0, CPW, body, 0)

        plsc.subcore_barrier()
        _writeout(s_sh, s_out, c, s)

    return k(y, eb, src1d, dst1d)


def _tc_eb(ea_pad, wc, bc):
    BE = 1024

    def body(ea_ref, w_ref, b_ref, *outs):
        i = pl.program_id(0)
        rid = lax.broadcasted_iota(jnp.int32, (BE, 1), 0) + i * BE
        valid = rid < E
        z = jnp.dot(ea_ref[...], w_ref[...],
                    preferred_element_type=jnp.float32) + b_ref[...]
        outs[0][...] = jnp.where(valid, jnp.maximum(z[:, :D], 0.0), 0.0)
        for l in range(1, 6):
            outs[l][...] = jnp.where(valid, z[:, l * D:(l + 1) * D], NEG)

    return pl.pallas_call(
        body,
        grid=(E2 // BE,),
        in_specs=[pl.BlockSpec((BE, ED), lambda i: (i, 0)),
                  pl.BlockSpec((ED, 6 * D), lambda i: (0, 0)),
                  pl.BlockSpec((1, 6 * D), lambda i: (0, 0))],
        out_specs=[pl.BlockSpec((BE, D), lambda i: (i, 0))] * 6,
        out_shape=[jax.ShapeDtypeStruct((E2, D), jnp.float32)] * 6,
    )(ea_pad, wc, bc)


_BN = 2000


def _tc_node0(s_p, c_p, w2, b2, w1t_next):
    def body(sp_ref, cp_ref, w2_ref, b2_ref, w1_ref,
             x_ref, y_ref, ci_ref, mk_ref):
        S = sp_ref[0] + sp_ref[1]
        cnt = jnp.max(cp_ref[0] + cp_ref[1], axis=1, keepdims=True)
        cinv = 1.0 / jnp.maximum(cnt, 1.0)
        mask = (cnt > 0.0).astype(jnp.float32)
        x0 = jnp.dot(S * cinv, w2_ref[...],
                     preferred_element_type=jnp.float32) + b2_ref[...] * mask
        x_ref[...] = x0
        y_ref[...] = jnp.dot(x0, w1_ref[...], preferred_element_type=jnp.float32)
        ci_ref[...] = jnp.broadcast_to(cinv, (_BN, D))
        mk_ref[...] = jnp.broadcast_to(mask, (_BN, D))

    return pl.pallas_call(
        body,
        grid=(N // _BN,),
        in_specs=[pl.BlockSpec((NC, _BN, D), lambda i: (0, i, 0)),
                  pl.BlockSpec((NC, _BN, D), lambda i: (0, i, 0)),
                  pl.BlockSpec((D, D), lambda i: (0, 0)),
                  pl.BlockSpec((1, D), lambda i: (0, 0)),
                  pl.BlockSpec((D, D), lambda i: (0, 0))],
        out_specs=[pl.BlockSpec((_BN, D), lambda i: (i, 0))] * 4,
        out_shape=[jax.ShapeDtypeStruct((N, D), jnp.float32)] * 4,
    )(s_p, c_p, w2, b2, w1t_next)


def _tc_node(s_p, cib, mkb, x_prev, wself, bself, w2, b2, w1t_next):
    def body(sp_ref, ci_ref, mk_ref, xp_ref, ws_ref, bs_ref, w2_ref, b2_ref,
             w1_ref, x_ref, y_ref):
        S = sp_ref[0] + sp_ref[1]
        aggr = jnp.dot(S * ci_ref[...], w2_ref[...],
                       preferred_element_type=jnp.float32) + b2_ref[...] * mk_ref[...]
        xs = jnp.dot(xp_ref[...], ws_ref[...],
                     preferred_element_type=jnp.float32) + bs_ref[...]
        x = jnp.maximum(xs + aggr, 0.0)
        x_ref[...] = x
        y_ref[...] = jnp.dot(x, w1_ref[...], preferred_element_type=jnp.float32)

    return pl.pallas_call(
        body,
        grid=(N // _BN,),
        in_specs=[pl.BlockSpec((NC, _BN, D), lambda i: (0, i, 0)),
                  pl.BlockSpec((_BN, D), lambda i: (i, 0)),
                  pl.BlockSpec((_BN, D), lambda i: (i, 0)),
                  pl.BlockSpec((_BN, D), lambda i: (i, 0)),
                  pl.BlockSpec((D, D), lambda i: (0, 0)),
                  pl.BlockSpec((1, D), lambda i: (0, 0)),
                  pl.BlockSpec((D, D), lambda i: (0, 0)),
                  pl.BlockSpec((1, D), lambda i: (0, 0)),
                  pl.BlockSpec((D, D), lambda i: (0, 0))],
        out_specs=[pl.BlockSpec((_BN, D), lambda i: (i, 0))] * 2,
        out_shape=[jax.ShapeDtypeStruct((N, D), jnp.float32)] * 2,
    )(s_p, cib, mkb, x_prev, wself, bself, w2, b2, w1t_next)


def _tc_final(s_p, cib, mkb, x_prev, wself, bself, w2, b2,
              rw1, rb1, rw2t, rb2):
    def body(sp_ref, ci_ref, mk_ref, xp_ref, ws_ref, bs_ref, w2_ref, b2_ref,
             rw1_ref, rb1_ref, rw2_ref, rb2_ref, o_ref):
        S = sp_ref[0] + sp_ref[1]
        aggr = jnp.dot(S * ci_ref[...], w2_ref[...],
                       preferred_element_type=jnp.float32) + b2_ref[...] * mk_ref[...]
        x5 = jnp.dot(xp_ref[...], ws_ref[...],
                     preferred_element_type=jnp.float32) + bs_ref[...] + aggr
        h = jnp.maximum(jnp.dot(x5, rw1_ref[...],
                                preferred_element_type=jnp.float32) + rb1_ref[...], 0.0)
        o_ref[...] = (jnp.sum(h * rw2_ref[...], axis=1, keepdims=True)
                      + rb2_ref[...])

    return pl.pallas_call(
        body,
        grid=(N // _BN,),
        in_specs=[pl.BlockSpec((NC, _BN, D), lambda i: (0, i, 0)),
                  pl.BlockSpec((_BN, D), lambda i: (i, 0)),
                  pl.BlockSpec((_BN, D), lambda i: (i, 0)),
                  pl.BlockSpec((_BN, D), lambda i: (i, 0)),
                  pl.BlockSpec((D, D), lambda i: (0, 0)),
                  pl.BlockSpec((1, D), lambda i: (0, 0)),
                  pl.BlockSpec((D, D), lambda i: (0, 0)),
                  pl.BlockSpec((1, D), lambda i: (0, 0)),
                  pl.BlockSpec((D, D), lambda i: (0, 0)),
                  pl.BlockSpec((1, D), lambda i: (0, 0)),
                  pl.BlockSpec((1, D), lambda i: (0, 0)),
                  pl.BlockSpec((1, 1), lambda i: (0, 0))],
        out_specs=pl.BlockSpec((_BN, 1), lambda i: (i, 0)),
        out_shape=jax.ShapeDtypeStruct((N, 1), jnp.float32),
    )(s_p, cib, mkb, x_prev, wself, bself, w2, b2, rw1, rb1, rw2t, rb2)


def kernel(edge_index, edge_attr, params):
    p = params
    src = edge_index[0]
    dst = edge_index[1]
    pad = E2 - E

    ea_pad = jnp.pad(edge_attr, ((0, pad), (0, 0)))
    src1d = jnp.pad(src, (0, pad))
    dst1d = jnp.pad(dst, (0, pad))
    dstc1d = jnp.pad(dst, (0, pad), constant_values=N)

    wc = jnp.concatenate(
        [p['e2n_W1']] + [p['l%d_W1' % l][D:] for l in range(1, 6)], axis=1)
    bc = jnp.concatenate(
        [p['e2n_b1']] + [p['l%d_b1' % l] for l in range(1, 6)])[None, :]

    ebs = _tc_eb(ea_pad, wc, bc)

    c0p = _sc_counts(dstc1d)
    s0p = _sc_pass0(ebs[0], dst1d, c0p)
    x, y, cib, mkb = _tc_node0(s0p, c0p, p['e2n_W2'], p['e2n_b2'][None],
                               p['l1_W1'][:D])
    for l in range(1, 5):
        sp = _sc_pass(y, ebs[l], src1d, dst1d)
        x, y = _tc_node(sp, cib, mkb, x,
                        p['l%d_Wself' % l], p['l%d_bself' % l][None],
                        p['l%d_W2' % l], p['l%d_b2' % l][None],
                        p['l%d_W1' % (l + 1)][:D])
    sp = _sc_pass(y, ebs[5], src1d, dst1d)
    return _tc_final(sp, cib, mkb, x,
                     p['l5_Wself'], p['l5_bself'][None],
                     p['l5_W2'], p['l5_b2'][None],
                     p['reg_W1'], p['reg_b1'][None],
                     p['reg_W2'].T, p['reg_b2'][None])


# speedup vs baseline: 1.9217x; 1.4266x over previous
import functools

import jax
import jax.numpy as jnp
from jax import lax
from jax.experimental import pallas as pl
from jax.experimental.pallas import tpu as pltpu
from jax.experimental.pallas import tpu_sc as plsc

N = 10000
E = 320000
D = 128
ED = 16
NC = 2
NS = 16
NW = NC * NS
CHUNK = 128
CPW = 80
NCH2 = NW * CPW
E2 = NCH2 * CHUNK
NEG = -1e30
RPT = 624
REM = N - NS * RPT


@functools.cache
def _mesh():
    return plsc.VectorSubcoreMesh(
        core_axis_name="c", subcore_axis_name="s", num_cores=NC, num_subcores=NS)


def _zero_spmem(zsrc, sh, s):
    r0 = s * RPT
    for t in range(4):
        pltpu.sync_copy(zsrc, sh.at[pl.ds(r0 + t * CHUNK, CHUNK)])
    pltpu.sync_copy(zsrc.at[pl.ds(0, RPT - 4 * CHUNK)],
                    sh.at[pl.ds(r0 + 4 * CHUNK, RPT - 4 * CHUNK)])

    @pl.when(s == NS - 1)
    def _():
        pltpu.sync_copy(zsrc.at[pl.ds(0, REM)], sh.at[pl.ds(NS * RPT, REM)])


def _writeout(sh, out, c, s):
    r0 = s * RPT
    pltpu.sync_copy(sh.at[pl.ds(r0, RPT)], out.at[c].at[pl.ds(r0, RPT)])

    @pl.when(s == NS - 1)
    def _():
        pltpu.sync_copy(sh.at[pl.ds(NS * RPT, REM)],
                        out.at[c].at[pl.ds(NS * RPT, REM)])


def _fill_const(buf, nrows, val):
    v = jnp.full((16,), val, jnp.float32)

    def body(i, _):
        for j in range(D // 16):
            buf[i, pl.ds(j * 16, 16)] = v
        return 0
    lax.fori_loop(0, nrows, body, 0)


def _sc_counts(dstc1d):
    @functools.partial(
        pl.kernel,
        out_type=pltpu.HBM((NC, N, D), jnp.float32),
        mesh=_mesh(),
        scratch_types=[
            pltpu.VMEM((CHUNK,), jnp.int32),
            pltpu.VMEM((CHUNK, D), jnp.float32),
            pltpu.VMEM((CHUNK, D), jnp.float32),
            pltpu.VMEM_SHARED((N + 16, D), jnp.float32),
        ],
    )
    def k(dst_hbm, c_out, idx_d, ones, zc, c_sh):
        c = lax.axis_index("c")
        s = lax.axis_index("s")
        w = s * NC + c
        lo = w * CPW

        _fill_const(ones, CHUNK, 1.0)
        _fill_const(zc, CHUNK, 0.0)
        _zero_spmem(zc, c_sh, s)
        plsc.subcore_barrier()

        def cb(g, _):
            pltpu.sync_copy(dst_hbm.at[pl.ds((lo + g) * CHUNK, CHUNK)], idx_d)
            pltpu.sync_copy(ones, c_sh.at[idx_d], add=True)
            return 0
        lax.fori_loop(0, CPW, cb, 0)

        plsc.subcore_barrier()
        _writeout(c_sh, c_out, c, s)

    return k(dstc1d)


def _sc_pass0(eb0, dst1d, dep):
    @functools.partial(
        pl.kernel,
        out_type=pltpu.HBM((NC, N, D), jnp.float32),
        mesh=_mesh(),
        scratch_types=[
            pltpu.VMEM((CHUNK,), jnp.int32),
            pltpu.VMEM((2 * CHUNK, D), jnp.float32),
            pltpu.VMEM_SHARED((N, D), jnp.float32),
            pltpu.SemaphoreType.DMA,
        ],
    )
    def k(eb_hbm, dst_hbm, dep_hbm, s_out, idx_d, ebuf, s_sh, esem):
        c = lax.axis_index("c")
        s = lax.axis_index("s")
        w = s * NC + c
        lo = w * CPW

        _fill_const(ebuf, CHUNK, 0.0)
        _zero_spmem(ebuf.at[pl.ds(0, CHUNK)], s_sh, s)
        plsc.subcore_barrier()

        def issue(g, b):
            pltpu.async_copy(eb_hbm.at[pl.ds((lo + g) * CHUNK, CHUNK)],
                             ebuf.at[pl.ds(b * CHUNK, CHUNK)], esem)

        issue(0, 0)

        def body(g, _):
            b = lax.rem(g, 2)
            pltpu.sync_copy(dst_hbm.at[pl.ds((lo + g) * CHUNK, CHUNK)], idx_d)
            pltpu.make_async_copy(eb_hbm.at[pl.ds((lo + g) * CHUNK, CHUNK)],
                                  ebuf.at[pl.ds(b * CHUNK, CHUNK)],
                                  esem).wait()

            @pl.when(g + 1 < CPW)
            def _():
                issue(g + 1, 1 - b)
            pltpu.sync_copy(ebuf.at[pl.ds(b * CHUNK, CHUNK)],
                            s_sh.at[idx_d], add=True)
            return 0
        lax.fori_loop(0, CPW, body, 0)

        plsc.subcore_barrier()
        _writeout(s_sh, s_out, c, s)

    return k(eb0, dst1d, dep)


def _sc_pass(y, eb, src1d, dst1d):
    @functools.partial(
        pl.kernel,
        out_type=pltpu.HBM((NC, N, D), jnp.float32),
        mesh=_mesh(),
        scratch_types=[
            pltpu.VMEM((CHUNK,), jnp.int32),
            pltpu.VMEM((CHUNK,), jnp.int32),
            pltpu.VMEM((CHUNK, D), jnp.float32),
            pltpu.VMEM((2 * CHUNK, D), jnp.float32),
            pltpu.VMEM_SHARED((N, D), jnp.float32),
            pltpu.SemaphoreType.DMA,
            pltpu.SemaphoreType.DMA,
        ],
    )
    def k(y_hbm, eb_hbm, src_hbm, dst_hbm, s_out,
          idx_g, idx_d, rows, ebuf, s_sh, gsem, esem):
        c = lax.axis_index("c")
        s = lax.axis_index("s")
        w = s * NC + c
        lo = w * CPW

        _fill_const(ebuf, CHUNK, 0.0)
        _zero_spmem(ebuf.at[pl.ds(0, CHUNK)], s_sh, s)
        plsc.subcore_barrier()

        pltpu.async_copy(eb_hbm.at[pl.ds(lo * CHUNK, CHUNK)],
                         ebuf.at[pl.ds(0, CHUNK)], esem)

        def body(g, _):
            b = lax.rem(g, 2)
            r0b = b * CHUNK
            pltpu.sync_copy(src_hbm.at[pl.ds((lo + g) * CHUNK, CHUNK)], idx_g)
            pltpu.sync_copy(dst_hbm.at[pl.ds((lo + g) * CHUNK, CHUNK)], idx_d)
            pltpu.async_copy(y_hbm.at[idx_g], rows, gsem)
            pltpu.make_async_copy(y_hbm.at[idx_g], rows, gsem).wait()
            pltpu.make_async_copy(eb_hbm.at[pl.ds((lo + g) * CHUNK, CHUNK)],
                                  ebuf.at[pl.ds(r0b, CHUNK)], esem).wait()

            def mk_rbody(base):
                def rbody(i, _):
                    for j in range(D // 16):
                        sl = pl.ds(j * 16, 16)
                        rows[i, sl] = jnp.maximum(
                            rows[i, sl] + ebuf[base + i, sl], 0.0)
                    return 0
                return rbody

            @pl.when(b == 0)
            def _():
                lax.fori_loop(0, CHUNK, mk_rbody(0), 0)

            @pl.when(b == 1)
            def _():
                lax.fori_loop(0, CHUNK, mk_rbody(CHUNK), 0)

            pltpu.sync_copy(rows, s_sh.at[idx_d], add=True)

            @pl.when(g + 1 < CPW)
            def _():
                pltpu.async_copy(eb_hbm.at[pl.ds((lo + g + 1) * CHUNK, CHUNK)],
                                 ebuf.at[pl.ds((1 - b) * CHUNK, CHUNK)], esem)
            return 0
        lax.fori_loop(0, CPW, body, 0)

        plsc.subcore_barrier()
        _writeout(s_sh, s_out, c, s)

    return k(y, eb, src1d, dst1d)


def _tc_eb(ea_pad, wc, bc):
    BE = 1024

    def body(ea_ref, w_ref, b_ref, *outs):
        i = pl.program_id(0)
        rid = lax.broadcasted_iota(jnp.int32, (BE, 1), 0) + i * BE
        valid = rid < E
        z = jnp.dot(ea_ref[...], w_ref[...],
                    preferred_element_type=jnp.float32) + b_ref[...]
        outs[0][...] = jnp.where(valid, jnp.maximum(z[:, :D], 0.0), 0.0)
        for l in range(1, 6):
            outs[l][...] = jnp.where(valid, z[:, l * D:(l + 1) * D], NEG)

    return pl.pallas_call(
        body,
        grid=(E2 // BE,),
        in_specs=[pl.BlockSpec((BE, ED), lambda i: (i, 0)),
                  pl.BlockSpec((ED, 6 * D), lambda i: (0, 0)),
                  pl.BlockSpec((1, 6 * D), lambda i: (0, 0))],
        out_specs=[pl.BlockSpec((BE, D), lambda i: (i, 0))] * 6,
        out_shape=[jax.ShapeDtypeStruct((E2, D), jnp.float32)] * 6,
    )(ea_pad, wc, bc)


_BN = 2000


def _tc_node0(s_p, c_p, w2, b2, w1t_next):
    def body(sp_ref, cp_ref, w2_ref, b2_ref, w1_ref,
             x_ref, y_ref, ci_ref, mk_ref):
        S = sp_ref[0] + sp_ref[1]
        cnt = jnp.max(cp_ref[0] + cp_ref[1], axis=1, keepdims=True)
        cinv = 1.0 / jnp.maximum(cnt, 1.0)
        mask = (cnt > 0.0).astype(jnp.float32)
        x0 = jnp.dot(S * cinv, w2_ref[...],
                     preferred_element_type=jnp.float32) + b2_ref[...] * mask
        x_ref[...] = x0
        y_ref[...] = jnp.dot(x0, w1_ref[...], preferred_element_type=jnp.float32)
        ci_ref[...] = jnp.broadcast_to(cinv, (_BN, D))
        mk_ref[...] = jnp.broadcast_to(mask, (_BN, D))

    return pl.pallas_call(
        body,
        grid=(N // _BN,),
        in_specs=[pl.BlockSpec((NC, _BN, D), lambda i: (0, i, 0)),
                  pl.BlockSpec((NC, _BN, D), lambda i: (0, i, 0)),
                  pl.BlockSpec((D, D), lambda i: (0, 0)),
                  pl.BlockSpec((1, D), lambda i: (0, 0)),
                  pl.BlockSpec((D, D), lambda i: (0, 0))],
        out_specs=[pl.BlockSpec((_BN, D), lambda i: (i, 0))] * 4,
        out_shape=[jax.ShapeDtypeStruct((N, D), jnp.float32)] * 4,
    )(s_p, c_p, w2, b2, w1t_next)


def _tc_node(s_p, cib, mkb, x_prev, wself, bself, w2, b2, w1t_next):
    def body(sp_ref, ci_ref, mk_ref, xp_ref, ws_ref, bs_ref, w2_ref, b2_ref,
             w1_ref, x_ref, y_ref):
        S = sp_ref[0] + sp_ref[1]
        aggr = jnp.dot(S * ci_ref[...], w2_ref[...],
                       preferred_element_type=jnp.float32) + b2_ref[...] * mk_ref[...]
        xs = jnp.dot(xp_ref[...], ws_ref[...],
                     preferred_element_type=jnp.float32) + bs_ref[...]
        x = jnp.maximum(xs + aggr, 0.0)
        x_ref[...] = x
        y_ref[...] = jnp.dot(x, w1_ref[...], preferred_element_type=jnp.float32)

    return pl.pallas_call(
        body,
        grid=(N // _BN,),
        in_specs=[pl.BlockSpec((NC, _BN, D), lambda i: (0, i, 0)),
                  pl.BlockSpec((_BN, D), lambda i: (i, 0)),
                  pl.BlockSpec((_BN, D), lambda i: (i, 0)),
                  pl.BlockSpec((_BN, D), lambda i: (i, 0)),
                  pl.BlockSpec((D, D), lambda i: (0, 0)),
                  pl.BlockSpec((1, D), lambda i: (0, 0)),
                  pl.BlockSpec((D, D), lambda i: (0, 0)),
                  pl.BlockSpec((1, D), lambda i: (0, 0)),
                  pl.BlockSpec((D, D), lambda i: (0, 0))],
        out_specs=[pl.BlockSpec((_BN, D), lambda i: (i, 0))] * 2,
        out_shape=[jax.ShapeDtypeStruct((N, D), jnp.float32)] * 2,
    )(s_p, cib, mkb, x_prev, wself, bself, w2, b2, w1t_next)


def _tc_final(s_p, cib, mkb, x_prev, wself, bself, w2, b2,
              rw1, rb1, rw2t, rb2):
    def body(sp_ref, ci_ref, mk_ref, xp_ref, ws_ref, bs_ref, w2_ref, b2_ref,
             rw1_ref, rb1_ref, rw2_ref, rb2_ref, o_ref):
        S = sp_ref[0] + sp_ref[1]
        aggr = jnp.dot(S * ci_ref[...], w2_ref[...],
                       preferred_element_type=jnp.float32) + b2_ref[...] * mk_ref[...]
        x5 = jnp.dot(xp_ref[...], ws_ref[...],
                     preferred_element_type=jnp.float32) + bs_ref[...] + aggr
        h = jnp.maximum(jnp.dot(x5, rw1_ref[...],
                                preferred_element_type=jnp.float32) + rb1_ref[...], 0.0)
        o_ref[...] = (jnp.sum(h * rw2_ref[...], axis=1, keepdims=True)
                      + rb2_ref[...])

    return pl.pallas_call(
        body,
        grid=(N // _BN,),
        in_specs=[pl.BlockSpec((NC, _BN, D), lambda i: (0, i, 0)),
                  pl.BlockSpec((_BN, D), lambda i: (i, 0)),
                  pl.BlockSpec((_BN, D), lambda i: (i, 0)),
                  pl.BlockSpec((_BN, D), lambda i: (i, 0)),
                  pl.BlockSpec((D, D), lambda i: (0, 0)),
                  pl.BlockSpec((1, D), lambda i: (0, 0)),
                  pl.BlockSpec((D, D), lambda i: (0, 0)),
                  pl.BlockSpec((1, D), lambda i: (0, 0)),
                  pl.BlockSpec((D, D), lambda i: (0, 0)),
                  pl.BlockSpec((1, D), lambda i: (0, 0)),
                  pl.BlockSpec((1, D), lambda i: (0, 0)),
                  pl.BlockSpec((1, 1), lambda i: (0, 0))],
        out_specs=pl.BlockSpec((_BN, 1), lambda i: (i, 0)),
        out_shape=jax.ShapeDtypeStruct((N, 1), jnp.float32),
    )(s_p, cib, mkb, x_prev, wself, bself, w2, b2, rw1, rb1, rw2t, rb2)


def kernel(edge_index, edge_attr, params):
    p = params
    src = edge_index[0]
    dst = edge_index[1]
    pad = E2 - E

    ea_pad = jnp.pad(edge_attr, ((0, pad), (0, 0)))
    src1d = jnp.pad(src, (0, pad))
    dst1d = jnp.pad(dst, (0, pad))
    dstc1d = jnp.pad(dst, (0, pad), constant_values=N)

    wc = jnp.concatenate(
        [p['e2n_W1']] + [p['l%d_W1' % l][D:] for l in range(1, 6)], axis=1)
    bc = jnp.concatenate(
        [p['e2n_b1']] + [p['l%d_b1' % l] for l in range(1, 6)])[None, :]

    ebs = _tc_eb(ea_pad, wc, bc)

    c0p = _sc_counts(dstc1d)
    s0p = _sc_pass0(ebs[0], dst1d, c0p)
    x, y, cib, mkb = _tc_node0(s0p, c0p, p['e2n_W2'], p['e2n_b2'][None],
                               p['l1_W1'][:D])
    for l in range(1, 5):
        sp = _sc_pass(y, ebs[l], src1d, dst1d)
        x, y = _tc_node(sp, cib, mkb, x,
                        p['l%d_Wself' % l], p['l%d_bself' % l][None],
                        p['l%d_W2' % l], p['l%d_b2' % l][None],
                        p['l%d_W1' % (l + 1)][:D])
    sp = _sc_pass(y, ebs[5], src1d, dst1d)
    return _tc_final(sp, cib, mkb, x,
                     p['l5_Wself'], p['l5_bself'][None],
                     p['l5_W2'], p['l5_b2'][None],
                     p['reg_W1'], p['reg_b1'][None],
                     p['reg_W2'].T, p['reg_b2'][None])


# speedup vs baseline: 3.2134x; 1.6721x over previous
import functools

import jax
import jax.numpy as jnp
from jax import lax
from jax.experimental import pallas as pl
from jax.experimental.pallas import tpu as pltpu
from jax.experimental.pallas import tpu_sc as plsc

N = 10000
E = 320000
D = 128
ED = 16
NC = 2
NS = 16
NW = NC * NS
CHUNK = 128
CPW = 80
NCH2 = NW * CPW
E2 = NCH2 * CHUNK
NEG = -1e30
RPT = 624
REM = N - NS * RPT


@functools.cache
def _mesh():
    return plsc.VectorSubcoreMesh(
        core_axis_name="c", subcore_axis_name="s", num_cores=NC, num_subcores=NS)


def _zero_spmem(zsrc, sh, s):
    r0 = s * RPT
    for t in range(4):
        pltpu.sync_copy(zsrc, sh.at[pl.ds(r0 + t * CHUNK, CHUNK)])
    pltpu.sync_copy(zsrc.at[pl.ds(0, RPT - 4 * CHUNK)],
                    sh.at[pl.ds(r0 + 4 * CHUNK, RPT - 4 * CHUNK)])

    @pl.when(s == NS - 1)
    def _():
        pltpu.sync_copy(zsrc.at[pl.ds(0, REM)], sh.at[pl.ds(NS * RPT, REM)])


def _writeout(sh, out, c, s):
    r0 = s * RPT
    pltpu.sync_copy(sh.at[pl.ds(r0, RPT)], out.at[c].at[pl.ds(r0, RPT)])

    @pl.when(s == NS - 1)
    def _():
        pltpu.sync_copy(sh.at[pl.ds(NS * RPT, REM)],
                        out.at[c].at[pl.ds(NS * RPT, REM)])


def _fill_const(buf, nrows, val):
    v = jnp.full((16,), val, jnp.float32)

    def body(i, _):
        for j in range(D // 16):
            buf[i, pl.ds(j * 16, 16)] = v
        return 0
    lax.fori_loop(0, nrows, body, 0)


def _chunk_range(w):
    lo = (w * (E // CHUNK)) // NW
    hi = ((w + 1) * (E // CHUNK)) // NW
    return lo, hi


def _sc_counts(dstc1d):
    @functools.partial(
        pl.kernel,
        out_type=pltpu.HBM((NC, N, D), jnp.float32),
        mesh=_mesh(),
        scratch_types=[
            pltpu.VMEM((CHUNK,), jnp.int32),
            pltpu.VMEM((CHUNK, D), jnp.float32),
            pltpu.VMEM((CHUNK, D), jnp.float32),
            pltpu.VMEM_SHARED((N + 16, D), jnp.float32),
        ],
    )
    def k(dst_hbm, c_out, idx_d, ones, zc, c_sh):
        c = lax.axis_index("c")
        s = lax.axis_index("s")
        w = s * NC + c
        lo = w * CPW

        _fill_const(ones, CHUNK, 1.0)
        _fill_const(zc, CHUNK, 0.0)
        _zero_spmem(zc, c_sh, s)
        plsc.subcore_barrier()

        def cb(g, _):
            pltpu.sync_copy(dst_hbm.at[pl.ds((lo + g) * CHUNK, CHUNK)], idx_d)
            pltpu.sync_copy(ones, c_sh.at[idx_d], add=True)
            return 0
        lax.fori_loop(0, CPW, cb, 0)

        plsc.subcore_barrier()
        _writeout(c_sh, c_out, c, s)

    return k(dstc1d)


def _sc_pass0(eb0, dst1d, dep):
    @functools.partial(
        pl.kernel,
        out_type=pltpu.HBM((NC, N, D), jnp.float32),
        mesh=_mesh(),
        scratch_types=[
            pltpu.VMEM((CHUNK,), jnp.int32),
            pltpu.VMEM((CHUNK, D), jnp.float32),
            pltpu.VMEM_SHARED((N, D), jnp.float32),
        ],
    )
    def k(eb_hbm, dst_hbm, dep_hbm, s_out, idx_d, msg, s_sh):
        c = lax.axis_index("c")
        s = lax.axis_index("s")
        w = s * NC + c

        _fill_const(msg, CHUNK, 0.0)
        _zero_spmem(msg, s_sh, s)
        plsc.subcore_barrier()

        lo, hi = _chunk_range(w)

        def chunk_body(g, _):
            e0 = g * CHUNK
            pltpu.sync_copy(dst_hbm.at[pl.ds(e0, CHUNK)], idx_d)
            pltpu.sync_copy(eb_hbm.at[pl.ds(e0, CHUNK)], msg)
            pltpu.sync_copy(msg, s_sh.at[idx_d], add=True)
            return 0
        lax.fori_loop(lo, hi, chunk_body, 0)

        plsc.subcore_barrier()
        _writeout(s_sh, s_out, c, s)

    return k(eb0, dst1d, dep)


def _sc_pass(y, eb, src1d, dst1d):
    @functools.partial(
        pl.kernel,
        out_type=pltpu.HBM((NC, N, D), jnp.float32),
        mesh=_mesh(),
        scratch_types=[
            pltpu.VMEM((CHUNK,), jnp.int32),
            pltpu.VMEM((CHUNK,), jnp.int32),
            pltpu.VMEM((CHUNK, D), jnp.float32),
            pltpu.VMEM((CHUNK, D), jnp.float32),
            pltpu.VMEM_SHARED((N, D), jnp.float32),
            pltpu.SemaphoreType.DMA,
        ],
    )
    def k(y_hbm, eb_hbm, src_hbm, dst_hbm, s_out,
          idx_s, idx_d, rows, ebuf, s_sh, gsem):
        c = lax.axis_index("c")
        s = lax.axis_index("s")
        w = s * NC + c

        _fill_const(ebuf, CHUNK, 0.0)
        _zero_spmem(ebuf, s_sh, s)
        plsc.subcore_barrier()

        lo, hi = _chunk_range(w)

        def chunk_body(g, _):
            e0 = g * CHUNK
            pltpu.sync_copy(src_hbm.at[pl.ds(e0, CHUNK)], idx_s)
            pltpu.sync_copy(dst_hbm.at[pl.ds(e0, CHUNK)], idx_d)
            gath = pltpu.async_copy(y_hbm.at[idx_s], rows, gsem)
            pltpu.sync_copy(eb_hbm.at[pl.ds(e0, CHUNK)], ebuf)
            gath.wait()

            def rbody(i, _):
                for j in range(D // 16):
                    sl = pl.ds(j * 16, 16)
                    rows[i, sl] = jnp.maximum(rows[i, sl] + ebuf[i, sl], 0.0)
                return 0
            lax.fori_loop(0, CHUNK, rbody, 0)
            pltpu.sync_copy(rows, s_sh.at[idx_d], add=True)
            return 0
        lax.fori_loop(lo, hi, chunk_body, 0)

        plsc.subcore_barrier()
        _writeout(s_sh, s_out, c, s)

    return k(y, eb, src1d, dst1d)


def _tc_eb(ea_pad, wc, bc):
    BE = 1024

    def body(ea_ref, w_ref, b_ref, *outs):
        i = pl.program_id(0)
        rid = lax.broadcasted_iota(jnp.int32, (BE, 1), 0) + i * BE
        valid = rid < E
        z = jnp.dot(ea_ref[...], w_ref[...],
                    preferred_element_type=jnp.float32) + b_ref[...]
        outs[0][...] = jnp.where(valid, jnp.maximum(z[:, :D], 0.0), 0.0)
        for l in range(1, 6):
            outs[l][...] = jnp.where(valid, z[:, l * D:(l + 1) * D], NEG)

    return pl.pallas_call(
        body,
        grid=(E2 // BE,),
        in_specs=[pl.BlockSpec((BE, ED), lambda i: (i, 0)),
                  pl.BlockSpec((ED, 6 * D), lambda i: (0, 0)),
                  pl.BlockSpec((1, 6 * D), lambda i: (0, 0))],
        out_specs=[pl.BlockSpec((BE, D), lambda i: (i, 0))] * 6,
        out_shape=[jax.ShapeDtypeStruct((E2, D), jnp.float32)] * 6,
    )(ea_pad, wc, bc)


_BN = 2000


def _tc_node0(s_p, c_p, w2, b2, w1t_next):
    def body(sp_ref, cp_ref, w2_ref, b2_ref, w1_ref,
             x_ref, y_ref, ci_ref, mk_ref):
        S = sp_ref[0] + sp_ref[1]
        cnt = jnp.max(cp_ref[0] + cp_ref[1], axis=1, keepdims=True)
        cinv = 1.0 / jnp.maximum(cnt, 1.0)
        mask = (cnt > 0.0).astype(jnp.float32)
        x0 = jnp.dot(S * cinv, w2_ref[...],
                     preferred_element_type=jnp.float32) + b2_ref[...] * mask
        x_ref[...] = x0
        y_ref[...] = jnp.dot(x0, w1_ref[...], preferred_element_type=jnp.float32)
        ci_ref[...] = jnp.broadcast_to(cinv, (_BN, D))
        mk_ref[...] = jnp.broadcast_to(mask, (_BN, D))

    return pl.pallas_call(
        body,
        grid=(N // _BN,),
        in_specs=[pl.BlockSpec((NC, _BN, D), lambda i: (0, i, 0)),
                  pl.BlockSpec((NC, _BN, D), lambda i: (0, i, 0)),
                  pl.BlockSpec((D, D), lambda i: (0, 0)),
                  pl.BlockSpec((1, D), lambda i: (0, 0)),
                  pl.BlockSpec((D, D), lambda i: (0, 0))],
        out_specs=[pl.BlockSpec((_BN, D), lambda i: (i, 0))] * 4,
        out_shape=[jax.ShapeDtypeStruct((N, D), jnp.float32)] * 4,
    )(s_p, c_p, w2, b2, w1t_next)


def _tc_node(s_p, cib, mkb, x_prev, wself, bself, w2, b2, w1t_next):
    def body(sp_ref, ci_ref, mk_ref, xp_ref, ws_ref, bs_ref, w2_ref, b2_ref,
             w1_ref, x_ref, y_ref):
        S = sp_ref[0] + sp_ref[1]
        aggr = jnp.dot(S * ci_ref[...], w2_ref[...],
                       preferred_element_type=jnp.float32) + b2_ref[...] * mk_ref[...]
        xs = jnp.dot(xp_ref[...], ws_ref[...],
                     preferred_element_type=jnp.float32) + bs_ref[...]
        x = jnp.maximum(xs + aggr, 0.0)
        x_ref[...] = x
        y_ref[...] = jnp.dot(x, w1_ref[...], preferred_element_type=jnp.float32)

    return pl.pallas_call(
        body,
        grid=(N // _BN,),
        in_specs=[pl.BlockSpec((NC, _BN, D), lambda i: (0, i, 0)),
                  pl.BlockSpec((_BN, D), lambda i: (i, 0)),
                  pl.BlockSpec((_BN, D), lambda i: (i, 0)),
                  pl.BlockSpec((_BN, D), lambda i: (i, 0)),
                  pl.BlockSpec((D, D), lambda i: (0, 0)),
                  pl.BlockSpec((1, D), lambda i: (0, 0)),
                  pl.BlockSpec((D, D), lambda i: (0, 0)),
                  pl.BlockSpec((1, D), lambda i: (0, 0)),
                  pl.BlockSpec((D, D), lambda i: (0, 0))],
        out_specs=[pl.BlockSpec((_BN, D), lambda i: (i, 0))] * 2,
        out_shape=[jax.ShapeDtypeStruct((N, D), jnp.float32)] * 2,
    )(s_p, cib, mkb, x_prev, wself, bself, w2, b2, w1t_next)


def _tc_final(s_p, cib, mkb, x_prev, wself, bself, w2, b2,
              rw1, rb1, rw2t, rb2):
    def body(sp_ref, ci_ref, mk_ref, xp_ref, ws_ref, bs_ref, w2_ref, b2_ref,
             rw1_ref, rb1_ref, rw2_ref, rb2_ref, o_ref):
        S = sp_ref[0] + sp_ref[1]
        aggr = jnp.dot(S * ci_ref[...], w2_ref[...],
                       preferred_element_type=jnp.float32) + b2_ref[...] * mk_ref[...]
        x5 = jnp.dot(xp_ref[...], ws_ref[...],
                     preferred_element_type=jnp.float32) + bs_ref[...] + aggr
        h = jnp.maximum(jnp.dot(x5, rw1_ref[...],
                                preferred_element_type=jnp.float32) + rb1_ref[...], 0.0)
        o_ref[...] = (jnp.sum(h * rw2_ref[...], axis=1, keepdims=True)
                      + rb2_ref[...])

    return pl.pallas_call(
        body,
        grid=(N // _BN,),
        in_specs=[pl.BlockSpec((NC, _BN, D), lambda i: (0, i, 0)),
                  pl.BlockSpec((_BN, D), lambda i: (i, 0)),
                  pl.BlockSpec((_BN, D), lambda i: (i, 0)),
                  pl.BlockSpec((_BN, D), lambda i: (i, 0)),
                  pl.BlockSpec((D, D), lambda i: (0, 0)),
                  pl.BlockSpec((1, D), lambda i: (0, 0)),
                  pl.BlockSpec((D, D), lambda i: (0, 0)),
                  pl.BlockSpec((1, D), lambda i: (0, 0)),
                  pl.BlockSpec((D, D), lambda i: (0, 0)),
                  pl.BlockSpec((1, D), lambda i: (0, 0)),
                  pl.BlockSpec((1, D), lambda i: (0, 0)),
                  pl.BlockSpec((1, 1), lambda i: (0, 0))],
        out_specs=pl.BlockSpec((_BN, 1), lambda i: (i, 0)),
        out_shape=jax.ShapeDtypeStruct((N, 1), jnp.float32),
    )(s_p, cib, mkb, x_prev, wself, bself, w2, b2, rw1, rb1, rw2t, rb2)


def kernel(edge_index, edge_attr, params):
    p = params
    src = edge_index[0]
    dst = edge_index[1]
    pad = E2 - E

    ea_pad = jnp.pad(edge_attr, ((0, pad), (0, 0)))
    dstc1d = jnp.pad(dst, (0, pad), constant_values=N)

    wc = jnp.concatenate(
        [p['e2n_W1']] + [p['l%d_W1' % l][D:] for l in range(1, 6)], axis=1)
    bc = jnp.concatenate(
        [p['e2n_b1']] + [p['l%d_b1' % l] for l in range(1, 6)])[None, :]

    ebs = _tc_eb(ea_pad, wc, bc)

    c0p = _sc_counts(dstc1d)
    s0p = _sc_pass0(ebs[0], dst, c0p)
    x, y, cib, mkb = _tc_node0(s0p, c0p, p['e2n_W2'], p['e2n_b2'][None],
                               p['l1_W1'][:D])
    for l in range(1, 5):
        sp = _sc_pass(y, ebs[l], src, dst)
        x, y = _tc_node(sp, cib, mkb, x,
                        p['l%d_Wself' % l], p['l%d_bself' % l][None],
                        p['l%d_W2' % l], p['l%d_b2' % l][None],
                        p['l%d_W1' % (l + 1)][:D])
    sp = _sc_pass(y, ebs[5], src, dst)
    return _tc_final(sp, cib, mkb, x,
                     p['l5_Wself'], p['l5_bself'][None],
                     p['l5_W2'], p['l5_b2'][None],
                     p['reg_W1'], p['reg_b1'][None],
                     p['reg_W2'].T, p['reg_b2'][None])


# speedup vs baseline: 3.2152x; 1.0006x over previous
import functools

import jax
import jax.numpy as jnp
from jax import lax
from jax.experimental import pallas as pl
from jax.experimental.pallas import tpu as pltpu
from jax.experimental.pallas import tpu_sc as plsc

N = 10000
E = 320000
D = 128
ED = 16
NC = 2
NS = 16
NW = NC * NS
CHUNK = 128
CPW = 80
NCH2 = NW * CPW
E2 = NCH2 * CHUNK
NEG = -1e30
RPT = 624
REM = N - NS * RPT


@functools.cache
def _mesh():
    return plsc.VectorSubcoreMesh(
        core_axis_name="c", subcore_axis_name="s", num_cores=NC, num_subcores=NS)


def _zero_spmem(zsrc, sh, s):
    r0 = s * RPT
    for t in range(4):
        pltpu.sync_copy(zsrc, sh.at[pl.ds(r0 + t * CHUNK, CHUNK)])
    pltpu.sync_copy(zsrc.at[pl.ds(0, RPT - 4 * CHUNK)],
                    sh.at[pl.ds(r0 + 4 * CHUNK, RPT - 4 * CHUNK)])

    @pl.when(s == NS - 1)
    def _():
        pltpu.sync_copy(zsrc.at[pl.ds(0, REM)], sh.at[pl.ds(NS * RPT, REM)])


def _writeout(sh, out, c, s):
    r0 = s * RPT
    pltpu.sync_copy(sh.at[pl.ds(r0, RPT)], out.at[c].at[pl.ds(r0, RPT)])

    @pl.when(s == NS - 1)
    def _():
        pltpu.sync_copy(sh.at[pl.ds(NS * RPT, REM)],
                        out.at[c].at[pl.ds(NS * RPT, REM)])


def _fill_const(buf, nrows, val):
    v = jnp.full((16,), val, jnp.float32)

    def body(i, _):
        for j in range(D // 16):
            buf[i, pl.ds(j * 16, 16)] = v
        return 0
    lax.fori_loop(0, nrows, body, 0)


def _chunk_range(w):
    lo = (w * (E // CHUNK)) // NW
    hi = ((w + 1) * (E // CHUNK)) // NW
    return lo, hi


def _sc_counts(dstc1d):
    @functools.partial(
        pl.kernel,
        out_type=pltpu.HBM((NC, N, D), jnp.float32),
        mesh=_mesh(),
        scratch_types=[
            pltpu.VMEM((CHUNK,), jnp.int32),
            pltpu.VMEM((CHUNK, D), jnp.float32),
            pltpu.VMEM((CHUNK, D), jnp.float32),
            pltpu.VMEM_SHARED((N + 16, D), jnp.float32),
        ],
    )
    def k(dst_hbm, c_out, idx_d, ones, zc, c_sh):
        c = lax.axis_index("c")
        s = lax.axis_index("s")
        w = s * NC + c
        lo = w * CPW

        _fill_const(ones, CHUNK, 1.0)
        _fill_const(zc, CHUNK, 0.0)
        _zero_spmem(zc, c_sh, s)
        plsc.subcore_barrier()

        def cb(g, _):
            pltpu.sync_copy(dst_hbm.at[pl.ds((lo + g) * CHUNK, CHUNK)], idx_d)
            pltpu.sync_copy(ones, c_sh.at[idx_d], add=True)
            return 0
        lax.fori_loop(0, CPW, cb, 0)

        plsc.subcore_barrier()
        _writeout(c_sh, c_out, c, s)

    return k(dstc1d)


def _sc_pass0(eb0, dst1d, dep):
    @functools.partial(
        pl.kernel,
        out_type=pltpu.HBM((NC, N, D), jnp.float32),
        mesh=_mesh(),
        scratch_types=[
            pltpu.VMEM((CHUNK,), jnp.int32),
            pltpu.VMEM((CHUNK, D), jnp.float32),
            pltpu.VMEM_SHARED((N, D), jnp.float32),
        ],
    )
    def k(eb_hbm, dst_hbm, dep_hbm, s_out, idx_d, msg, s_sh):
        c = lax.axis_index("c")
        s = lax.axis_index("s")
        w = s * NC + c

        _fill_const(msg, CHUNK, 0.0)
        _zero_spmem(msg, s_sh, s)
        plsc.subcore_barrier()

        lo, hi = _chunk_range(w)

        def chunk_body(g, _):
            e0 = g * CHUNK
            pltpu.sync_copy(dst_hbm.at[pl.ds(e0, CHUNK)], idx_d)
            pltpu.sync_copy(eb_hbm.at[pl.ds(e0, CHUNK)], msg)
            pltpu.sync_copy(msg, s_sh.at[idx_d], add=True)
            return 0
        lax.fori_loop(lo, hi, chunk_body, 0)

        plsc.subcore_barrier()
        _writeout(s_sh, s_out, c, s)

    return k(eb0, dst1d, dep)


def _sc_pass(y, eb, src1d, dst1d):
    @functools.partial(
        pl.kernel,
        out_type=pltpu.HBM((NC, N, D), jnp.float32),
        mesh=_mesh(),
        scratch_types=[
            pltpu.VMEM((CHUNK,), jnp.int32),
            pltpu.VMEM((CHUNK,), jnp.int32),
            pltpu.VMEM((CHUNK, D), jnp.float32),
            pltpu.VMEM((CHUNK, D), jnp.float32),
            pltpu.VMEM_SHARED((N, D), jnp.float32),
            pltpu.SemaphoreType.DMA,
            pltpu.SemaphoreType.DMA,
        ],
    )
    def k(y_hbm, eb_hbm, src_hbm, dst_hbm, s_out,
          idx_s, idx_d, rows, ebuf, s_sh, gsem, ssem):
        c = lax.axis_index("c")
        s = lax.axis_index("s")
        w = s * NC + c

        _fill_const(ebuf, CHUNK, 0.0)
        _zero_spmem(ebuf, s_sh, s)
        plsc.subcore_barrier()

        lo, hi = _chunk_range(w)

        def chunk_body(g, _):
            e0 = g * CHUNK

            @pl.when(g > lo)
            def _():
                pltpu.make_async_copy(rows, s_sh.at[idx_d], ssem).wait()
            pltpu.sync_copy(src_hbm.at[pl.ds(e0, CHUNK)], idx_s)
            pltpu.sync_copy(dst_hbm.at[pl.ds(e0, CHUNK)], idx_d)
            gath = pltpu.async_copy(y_hbm.at[idx_s], rows, gsem)
            pltpu.sync_copy(eb_hbm.at[pl.ds(e0, CHUNK)], ebuf)
            gath.wait()

            def rbody(i, _):
                i2 = i * 2
                for j in range(D // 16):
                    sl = pl.ds(j * 16, 16)
                    rows[i2, sl] = jnp.maximum(rows[i2, sl] + ebuf[i2, sl], 0.0)
                for j in range(D // 16):
                    sl = pl.ds(j * 16, 16)
                    rows[i2 + 1, sl] = jnp.maximum(rows[i2 + 1, sl]
                                                   + ebuf[i2 + 1, sl], 0.0)
                return 0
            lax.fori_loop(0, CHUNK // 2, rbody, 0)
            pltpu.async_copy(rows, s_sh.at[idx_d], ssem, add=True)
            return 0
        lax.fori_loop(lo, hi, chunk_body, 0)
        pltpu.make_async_copy(rows, s_sh.at[idx_d], ssem).wait()

        plsc.subcore_barrier()
        _writeout(s_sh, s_out, c, s)

    return k(y, eb, src1d, dst1d)


def _tc_eb(ea_pad, wc, bc):
    BE = 1024

    def body(ea_ref, w_ref, b_ref, *outs):
        i = pl.program_id(0)
        rid = lax.broadcasted_iota(jnp.int32, (BE, 1), 0) + i * BE
        valid = rid < E
        z = jnp.dot(ea_ref[...], w_ref[...],
                    preferred_element_type=jnp.float32) + b_ref[...]
        outs[0][...] = jnp.where(valid, jnp.maximum(z[:, :D], 0.0), 0.0)
        for l in range(1, 6):
            outs[l][...] = jnp.where(valid, z[:, l * D:(l + 1) * D], NEG)

    return pl.pallas_call(
        body,
        grid=(E2 // BE,),
        in_specs=[pl.BlockSpec((BE, ED), lambda i: (i, 0)),
                  pl.BlockSpec((ED, 6 * D), lambda i: (0, 0)),
                  pl.BlockSpec((1, 6 * D), lambda i: (0, 0))],
        out_specs=[pl.BlockSpec((BE, D), lambda i: (i, 0))] * 6,
        out_shape=[jax.ShapeDtypeStruct((E2, D), jnp.float32)] * 6,
    )(ea_pad, wc, bc)


_BN = 2000


def _tc_node0(s_p, c_p, w2, b2, w1t_next):
    def body(sp_ref, cp_ref, w2_ref, b2_ref, w1_ref,
             x_ref, y_ref, ci_ref, mk_ref):
        S = sp_ref[0] + sp_ref[1]
        cnt = jnp.max(cp_ref[0] + cp_ref[1], axis=1, keepdims=True)
        cinv = 1.0 / jnp.maximum(cnt, 1.0)
        mask = (cnt > 0.0).astype(jnp.float32)
        x0 = jnp.dot(S * cinv, w2_ref[...],
                     preferred_element_type=jnp.float32) + b2_ref[...] * mask
        x_ref[...] = x0
        y_ref[...] = jnp.dot(x0, w1_ref[...], preferred_element_type=jnp.float32)
        ci_ref[...] = jnp.broadcast_to(cinv, (_BN, D))
        mk_ref[...] = jnp.broadcast_to(mask, (_BN, D))

    return pl.pallas_call(
        body,
        grid=(N // _BN,),
        in_specs=[pl.BlockSpec((NC, _BN, D), lambda i: (0, i, 0)),
                  pl.BlockSpec((NC, _BN, D), lambda i: (0, i, 0)),
                  pl.BlockSpec((D, D), lambda i: (0, 0)),
                  pl.BlockSpec((1, D), lambda i: (0, 0)),
                  pl.BlockSpec((D, D), lambda i: (0, 0))],
        out_specs=[pl.BlockSpec((_BN, D), lambda i: (i, 0))] * 4,
        out_shape=[jax.ShapeDtypeStruct((N, D), jnp.float32)] * 4,
    )(s_p, c_p, w2, b2, w1t_next)


def _tc_node(s_p, cib, mkb, x_prev, wself, bself, w2, b2, w1t_next):
    def body(sp_ref, ci_ref, mk_ref, xp_ref, ws_ref, bs_ref, w2_ref, b2_ref,
             w1_ref, x_ref, y_ref):
        S = sp_ref[0] + sp_ref[1]
        aggr = jnp.dot(S * ci_ref[...], w2_ref[...],
                       preferred_element_type=jnp.float32) + b2_ref[...] * mk_ref[...]
        xs = jnp.dot(xp_ref[...], ws_ref[...],
                     preferred_element_type=jnp.float32) + bs_ref[...]
        x = jnp.maximum(xs + aggr, 0.0)
        x_ref[...] = x
        y_ref[...] = jnp.dot(x, w1_ref[...], preferred_element_type=jnp.float32)

    return pl.pallas_call(
        body,
        grid=(N // _BN,),
        in_specs=[pl.BlockSpec((NC, _BN, D), lambda i: (0, i, 0)),
                  pl.BlockSpec((_BN, D), lambda i: (i, 0)),
                  pl.BlockSpec((_BN, D), lambda i: (i, 0)),
                  pl.BlockSpec((_BN, D), lambda i: (i, 0)),
                  pl.BlockSpec((D, D), lambda i: (0, 0)),
                  pl.BlockSpec((1, D), lambda i: (0, 0)),
                  pl.BlockSpec((D, D), lambda i: (0, 0)),
                  pl.BlockSpec((1, D), lambda i: (0, 0)),
                  pl.BlockSpec((D, D), lambda i: (0, 0))],
        out_specs=[pl.BlockSpec((_BN, D), lambda i: (i, 0))] * 2,
        out_shape=[jax.ShapeDtypeStruct((N, D), jnp.float32)] * 2,
    )(s_p, cib, mkb, x_prev, wself, bself, w2, b2, w1t_next)


def _tc_final(s_p, cib, mkb, x_prev, wself, bself, w2, b2,
              rw1, rb1, rw2t, rb2):
    def body(sp_ref, ci_ref, mk_ref, xp_ref, ws_ref, bs_ref, w2_ref, b2_ref,
             rw1_ref, rb1_ref, rw2_ref, rb2_ref, o_ref):
        S = sp_ref[0] + sp_ref[1]
        aggr = jnp.dot(S * ci_ref[...], w2_ref[...],
                       preferred_element_type=jnp.float32) + b2_ref[...] * mk_ref[...]
        x5 = jnp.dot(xp_ref[...], ws_ref[...],
                     preferred_element_type=jnp.float32) + bs_ref[...] + aggr
        h = jnp.maximum(jnp.dot(x5, rw1_ref[...],
                                preferred_element_type=jnp.float32) + rb1_ref[...], 0.0)
        o_ref[...] = (jnp.sum(h * rw2_ref[...], axis=1, keepdims=True)
                      + rb2_ref[...])

    return pl.pallas_call(
        body,
        grid=(N // _BN,),
        in_specs=[pl.BlockSpec((NC, _BN, D), lambda i: (0, i, 0)),
                  pl.BlockSpec((_BN, D), lambda i: (i, 0)),
                  pl.BlockSpec((_BN, D), lambda i: (i, 0)),
                  pl.BlockSpec((_BN, D), lambda i: (i, 0)),
                  pl.BlockSpec((D, D), lambda i: (0, 0)),
                  pl.BlockSpec((1, D), lambda i: (0, 0)),
                  pl.BlockSpec((D, D), lambda i: (0, 0)),
                  pl.BlockSpec((1, D), lambda i: (0, 0)),
                  pl.BlockSpec((D, D), lambda i: (0, 0)),
                  pl.BlockSpec((1, D), lambda i: (0, 0)),
                  pl.BlockSpec((1, D), lambda i: (0, 0)),
                  pl.BlockSpec((1, 1), lambda i: (0, 0))],
        out_specs=pl.BlockSpec((_BN, 1), lambda i: (i, 0)),
        out_shape=jax.ShapeDtypeStruct((N, 1), jnp.float32),
    )(s_p, cib, mkb, x_prev, wself, bself, w2, b2, rw1, rb1, rw2t, rb2)


def kernel(edge_index, edge_attr, params):
    p = params
    src = edge_index[0]
    dst = edge_index[1]
    pad = E2 - E

    ea_pad = jnp.pad(edge_attr, ((0, pad), (0, 0)))
    dstc1d = jnp.pad(dst, (0, pad), constant_values=N)

    wc = jnp.concatenate(
        [p['e2n_W1']] + [p['l%d_W1' % l][D:] for l in range(1, 6)], axis=1)
    bc = jnp.concatenate(
        [p['e2n_b1']] + [p['l%d_b1' % l] for l in range(1, 6)])[None, :]

    ebs = _tc_eb(ea_pad, wc, bc)

    c0p = _sc_counts(dstc1d)
    s0p = _sc_pass0(ebs[0], dst, c0p)
    x, y, cib, mkb = _tc_node0(s0p, c0p, p['e2n_W2'], p['e2n_b2'][None],
                               p['l1_W1'][:D])
    for l in range(1, 5):
        sp = _sc_pass(y, ebs[l], src, dst)
        x, y = _tc_node(sp, cib, mkb, x,
                        p['l%d_Wself' % l], p['l%d_bself' % l][None],
                        p['l%d_W2' % l], p['l%d_b2' % l][None],
                        p['l%d_W1' % (l + 1)][:D])
    sp = _sc_pass(y, ebs[5], src, dst)
    return _tc_final(sp, cib, mkb, x,
                     p['l5_Wself'], p['l5_bself'][None],
                     p['l5_W2'], p['l5_b2'][None],
                     p['reg_W1'], p['reg_b1'][None],
                     p['reg_W2'].T, p['reg_b2'][None])
